# baseline (speedup 1.0000x reference)
import jax
import jax.numpy as jnp
from jax import lax
from jax.experimental import pallas as pl
from jax.experimental.pallas import tpu as pltpu

HEADS = 16
BUCKETS = 64
SEQ = 4096
DIM = 128
TOK = SEQ // BUCKETS
SL = 4


def _sortnet(mq, mk, qpos, kpos, wq, wk):
    sq = (jnp.dot(mq, wq[:DIM], preferred_element_type=jnp.float32)
          + jnp.dot(qpos, wq[DIM:], preferred_element_type=jnp.float32))
    sk = (jnp.dot(mk, wk[:DIM], preferred_element_type=jnp.float32)
          + jnp.dot(kpos, wk[DIM:], preferred_element_type=jnp.float32))
    r = lax.dot_general(sq, sk, (((1,), (1,)), ((), ())),
                        preferred_element_type=jnp.float32)
    r = r - jnp.max(r, axis=-1, keepdims=True)
    e = jnp.exp(r)
    return e / jnp.sum(e, axis=-1, keepdims=True)


def _body(q_ref, k_ref, qpos_ref, kpos_ref, wq_ref, wk_ref, out_ref):
    for s in range(SL):
        mq = jnp.sum(q_ref[s].reshape(BUCKETS, TOK, DIM), axis=1) * (
            jnp.float32(1.0 / TOK))
        mk = jnp.sum(k_ref[s].reshape(BUCKETS, TOK, DIM), axis=1) * (
            jnp.float32(1.0 / TOK))
        out_ref[s] = _sortnet(mq, mk, qpos_ref[0, s], kpos_ref[0, s],
                              wq_ref[0, s], wk_ref[0, s])


def kernel(q, k, q_pos_emb, k_pos_emb, linear_sort_q, linear_sort_k):
    bh = q.shape[0]
    n = bh // SL
    return pl.pallas_call(
        _body,
        grid=(n,),
        in_specs=[
            pl.BlockSpec((SL, SEQ, DIM), lambda i: (i, 0, 0)),
            pl.BlockSpec((SL, SEQ, DIM), lambda i: (i, 0, 0)),
            pl.BlockSpec((1, SL, BUCKETS, DIM),
                         lambda i: (0, i % (HEADS // SL), 0, 0)),
            pl.BlockSpec((1, SL, BUCKETS, DIM),
                         lambda i: (0, i % (HEADS // SL), 0, 0)),
            pl.BlockSpec((1, SL, 2 * DIM, DIM),
                         lambda i: (0, i % (HEADS // SL), 0, 0)),
            pl.BlockSpec((1, SL, 2 * DIM, DIM),
                         lambda i: (0, i % (HEADS // SL), 0, 0)),
        ],
        out_specs=pl.BlockSpec((SL, BUCKETS, BUCKETS), lambda i: (i, 0, 0)),
        out_shape=jax.ShapeDtypeStruct((bh, BUCKETS, BUCKETS), jnp.float32),
    )(q, k, q_pos_emb, k_pos_emb, linear_sort_q, linear_sort_k)


# speedup vs baseline: 1.0284x; 1.0284x over previous
import jax
import jax.numpy as jnp
from jax import lax
from jax.experimental import pallas as pl
from jax.experimental.pallas import tpu as pltpu

HEADS = 16
BUCKETS = 64
SEQ = 4096
DIM = 128
TOK = SEQ // BUCKETS
SL = 2
NBUF = 3


def _sortnet(mq, mk, qpos, kpos, wq, wk):
    sq = (jnp.dot(mq, wq[:DIM], preferred_element_type=jnp.float32)
          + jnp.dot(qpos, wq[DIM:], preferred_element_type=jnp.float32))
    sk = (jnp.dot(mk, wk[:DIM], preferred_element_type=jnp.float32)
          + jnp.dot(kpos, wk[DIM:], preferred_element_type=jnp.float32))
    r = lax.dot_general(sq, sk, (((1,), (1,)), ((), ())),
                        preferred_element_type=jnp.float32)
    r = r - jnp.max(r, axis=-1, keepdims=True)
    e = jnp.exp(r)
    return e / jnp.sum(e, axis=-1, keepdims=True)


def _body(q_hbm, k_hbm, qpos_ref, kpos_ref, wq_ref, wk_ref, out_ref,
          qbuf, kbuf, qsem, ksem):
    nstep = out_ref.shape[0] // SL

    def start(i, slot):
        pltpu.make_async_copy(
            q_hbm.at[pl.ds(i * SL, SL)], qbuf.at[slot], qsem.at[slot]).start()
        pltpu.make_async_copy(
            k_hbm.at[pl.ds(i * SL, SL)], kbuf.at[slot], ksem.at[slot]).start()

    for i in range(NBUF):
        start(i, i)

    def step(i, _):
        slot = lax.rem(i, NBUF)
        pltpu.make_async_copy(
            q_hbm.at[pl.ds(i * SL, SL)], qbuf.at[slot], qsem.at[slot]).wait()
        pltpu.make_async_copy(
            k_hbm.at[pl.ds(i * SL, SL)], kbuf.at[slot], ksem.at[slot]).wait()
        for s in range(SL):
            h = lax.rem(i * SL + s, HEADS)
            mq = jnp.sum(qbuf[slot, s].reshape(BUCKETS, TOK, DIM), axis=1) * (
                jnp.float32(1.0 / TOK))
            mk = jnp.sum(kbuf[slot, s].reshape(BUCKETS, TOK, DIM), axis=1) * (
                jnp.float32(1.0 / TOK))
            out_ref[i * SL + s] = _sortnet(
                mq, mk, qpos_ref[0, h], kpos_ref[0, h],
                wq_ref[0, h], wk_ref[0, h])

        @pl.when(i + NBUF < nstep)
        def _():
            start(i + NBUF, slot)
        return 0

    lax.fori_loop(0, nstep, step, 0)


def kernel(q, k, q_pos_emb, k_pos_emb, linear_sort_q, linear_sort_k):
    bh = q.shape[0]
    return pl.pallas_call(
        _body,
        in_specs=[
            pl.BlockSpec(memory_space=pl.ANY),
            pl.BlockSpec(memory_space=pl.ANY),
            pl.BlockSpec((1, HEADS, BUCKETS, DIM), lambda: (0, 0, 0, 0)),
            pl.BlockSpec((1, HEADS, BUCKETS, DIM), lambda: (0, 0, 0, 0)),
            pl.BlockSpec((1, HEADS, 2 * DIM, DIM), lambda: (0, 0, 0, 0)),
            pl.BlockSpec((1, HEADS, 2 * DIM, DIM), lambda: (0, 0, 0, 0)),
        ],
        out_specs=pl.BlockSpec((bh, BUCKETS, BUCKETS), lambda: (0, 0, 0)),
        out_shape=jax.ShapeDtypeStruct((bh, BUCKETS, BUCKETS), jnp.float32),
        scratch_shapes=[
            pltpu.VMEM((NBUF, SL, SEQ, DIM), jnp.float32),
            pltpu.VMEM((NBUF, SL, SEQ, DIM), jnp.float32),
            pltpu.SemaphoreType.DMA((NBUF,)),
            pltpu.SemaphoreType.DMA((NBUF,)),
        ],
    )(q, k, q_pos_emb, k_pos_emb, linear_sort_q, linear_sort_k)


# speedup vs baseline: 1.0425x; 1.0137x over previous
import jax
import jax.numpy as jnp
from jax import lax
from jax.experimental import pallas as pl
from jax.experimental.pallas import tpu as pltpu

HEADS = 16
BUCKETS = 64
SEQ = 4096
DIM = 128
TOK = SEQ // BUCKETS
SL = 2


def _sortnet(mq, mk, qpos, kpos, wq, wk):
    sq = (jnp.dot(mq, wq[:DIM], preferred_element_type=jnp.float32)
          + jnp.dot(qpos, wq[DIM:], preferred_element_type=jnp.float32))
    sk = (jnp.dot(mk, wk[:DIM], preferred_element_type=jnp.float32)
          + jnp.dot(kpos, wk[DIM:], preferred_element_type=jnp.float32))
    r = lax.dot_general(sq, sk, (((1,), (1,)), ((), ())),
                        preferred_element_type=jnp.float32)
    r = r - jnp.max(r, axis=-1, keepdims=True)
    e = jnp.exp(r)
    return e / jnp.sum(e, axis=-1, keepdims=True)


def _body(q_ref, k_ref, qpos_ref, kpos_ref, wq_ref, wk_ref, out_ref):
    for s in range(SL):
        mq = jnp.sum(q_ref[s].reshape(BUCKETS, TOK, DIM), axis=1) * (
            jnp.float32(1.0 / TOK))
        mk = jnp.sum(k_ref[s].reshape(BUCKETS, TOK, DIM), axis=1) * (
            jnp.float32(1.0 / TOK))
        out_ref[s] = _sortnet(mq, mk, qpos_ref[0, s], kpos_ref[0, s],
                              wq_ref[0, s], wk_ref[0, s])


def kernel(q, k, q_pos_emb, k_pos_emb, linear_sort_q, linear_sort_k):
    bh = q.shape[0]
    n = bh // SL
    return pl.pallas_call(
        _body,
        grid=(n,),
        in_specs=[
            pl.BlockSpec((SL, SEQ, DIM), lambda i: (i, 0, 0)),
            pl.BlockSpec((SL, SEQ, DIM), lambda i: (i, 0, 0)),
            pl.BlockSpec((1, SL, BUCKETS, DIM),
                         lambda i: (0, i % (HEADS // SL), 0, 0)),
            pl.BlockSpec((1, SL, BUCKETS, DIM),
                         lambda i: (0, i % (HEADS // SL), 0, 0)),
            pl.BlockSpec((1, SL, 2 * DIM, DIM),
                         lambda i: (0, i % (HEADS // SL), 0, 0)),
            pl.BlockSpec((1, SL, 2 * DIM, DIM),
                         lambda i: (0, i % (HEADS // SL), 0, 0)),
        ],
        out_specs=pl.BlockSpec((SL, BUCKETS, BUCKETS), lambda i: (i, 0, 0)),
        out_shape=jax.ShapeDtypeStruct((bh, BUCKETS, BUCKETS), jnp.float32),
    )(q, k, q_pos_emb, k_pos_emb, linear_sort_q, linear_sort_k)


# speedup vs baseline: 1.0823x; 1.0382x over previous
import jax
import jax.numpy as jnp
from jax import lax
from jax.experimental import pallas as pl
from jax.experimental.pallas import tpu as pltpu

HEADS = 16
BUCKETS = 64
SEQ = 4096
DIM = 128
TOK = SEQ // BUCKETS
SL = 2


def _sortnet(mq, mk, qpos, kpos, wq, wk):
    sq = (jnp.dot(mq, wq[:DIM], preferred_element_type=jnp.float32)
          + jnp.dot(qpos, wq[DIM:], preferred_element_type=jnp.float32))
    sk = (jnp.dot(mk, wk[:DIM], preferred_element_type=jnp.float32)
          + jnp.dot(kpos, wk[DIM:], preferred_element_type=jnp.float32))
    r = lax.dot_general(sq, sk, (((1,), (1,)), ((), ())),
                        preferred_element_type=jnp.float32)
    r = r - jnp.max(r, axis=-1, keepdims=True)
    e = jnp.exp(r)
    return e / jnp.sum(e, axis=-1, keepdims=True)


def _body(q_ref, k_ref, qpos_ref, kpos_ref, wq_ref, wk_ref, out_ref):
    i = pl.program_id(0)
    for s in range(SL):
        h = lax.rem(i * SL + s, HEADS)
        mq = jnp.sum(q_ref[s].reshape(BUCKETS, TOK, DIM), axis=1) * (
            jnp.float32(1.0 / TOK))
        mk = jnp.sum(k_ref[s].reshape(BUCKETS, TOK, DIM), axis=1) * (
            jnp.float32(1.0 / TOK))
        out_ref[s] = _sortnet(mq, mk, qpos_ref[0, h], kpos_ref[0, h],
                              wq_ref[0, h], wk_ref[0, h])


def kernel(q, k, q_pos_emb, k_pos_emb, linear_sort_q, linear_sort_k):
    bh = q.shape[0]
    n = bh // SL
    return pl.pallas_call(
        _body,
        grid=(n,),
        in_specs=[
            pl.BlockSpec((SL, SEQ, DIM), lambda i: (i, 0, 0)),
            pl.BlockSpec((SL, SEQ, DIM), lambda i: (i, 0, 0)),
            pl.BlockSpec((1, HEADS, BUCKETS, DIM), lambda i: (0, 0, 0, 0)),
            pl.BlockSpec((1, HEADS, BUCKETS, DIM), lambda i: (0, 0, 0, 0)),
            pl.BlockSpec((1, HEADS, 2 * DIM, DIM), lambda i: (0, 0, 0, 0)),
            pl.BlockSpec((1, HEADS, 2 * DIM, DIM), lambda i: (0, 0, 0, 0)),
        ],
        out_specs=pl.BlockSpec((SL, BUCKETS, BUCKETS), lambda i: (i, 0, 0)),
        out_shape=jax.ShapeDtypeStruct((bh, BUCKETS, BUCKETS), jnp.float32),
    )(q, k, q_pos_emb, k_pos_emb, linear_sort_q, linear_sort_k)
